# baseline (speedup 1.0000x reference)
import functools

import jax
import jax.numpy as jnp
from jax import lax
from jax.experimental import pallas as pl
from jax.experimental.pallas import tpu as pltpu
from jax.experimental.pallas import tpu_sc as plsc

R = 100
P = 2048
LANES = 16
MV = 8
BATCH = MV * LANES


def _conv_body(x_ref, w_ref, o_ref):
    o_ref[...] = jnp.sum(x_ref[...] * w_ref[...], axis=1, keepdims=True)


def _conv_embed(x2, w2, block_rows):
    rows = x2.shape[0]
    grid = rows // block_rows
    return pl.pallas_call(
        _conv_body,
        grid=(grid,),
        in_specs=[
            pl.BlockSpec((block_rows, P), lambda i: (i, 0)),
            pl.BlockSpec((1, P), lambda i: (0, 0)),
        ],
        out_specs=pl.BlockSpec((block_rows, 1), lambda i: (i, 0)),
        out_shape=jax.ShapeDtypeStruct((rows, 1), jnp.float32),
    )(x2, w2)


def _s16(v, desc):
    k, _ = plsc.sort_key_val(v, v, descending=desc)
    return k


def _vmerge(vs, desc):
    if len(vs) == 1:
        return [_s16(vs[0], desc)]
    h = len(vs) // 2
    hi = [jnp.maximum(vs[i], vs[i + h]) for i in range(h)]
    lo = [jnp.minimum(vs[i], vs[i + h]) for i in range(h)]
    new = (hi + lo) if desc else (lo + hi)
    return _vmerge(new[:h], desc) + _vmerge(new[h:], desc)


def _bsort(vs, desc):
    if len(vs) == 1:
        return [_s16(vs[0], desc)]
    h = len(vs) // 2
    return _vmerge(_bsort(vs[:h], True) + _bsort(vs[h:], False), desc)


def _rev(vs):
    return [lax.rev(v, (0,)) for v in reversed(vs)]


def _topk_pool(emb):
    B, N = emb.shape
    out_w = 2 * MV * LANES
    mesh = plsc.VectorSubcoreMesh(core_axis_name="c", subcore_axis_name="s",
                                  num_cores=2, num_subcores=16)

    @functools.partial(
        pl.kernel,
        out_type=jax.ShapeDtypeStruct((B, out_w), jnp.float32),
        mesh=mesh,
        scratch_types=[
            pltpu.VMEM((N,), jnp.float32),
            pltpu.VMEM((out_w,), jnp.float32),
        ],
        compiler_params=pltpu.CompilerParams(needs_layout_passes=False),
    )
    def topk_kernel(emb_hbm, out_hbm, row_v, out_v):
        wid = lax.axis_index("s") * 2 + lax.axis_index("c")

        @pl.when(wid < B)
        def _():
            pltpu.sync_copy(emb_hbm.at[wid], row_v)

            def load(base):
                return [row_v[pl.ds(base + LANES * j, LANES)] for j in range(MV)]

            top0 = _bsort(load(0), True)
            bot0 = _rev(top0)

            def body(i, carry):
                top = list(carry[:MV])
                bot = list(carry[MV:])
                s_asc = _bsort(load(i * BATCH), False)
                s_desc = _rev(s_asc)
                top = _vmerge(
                    [jnp.maximum(top[j], s_asc[j]) for j in range(MV)], True)
                bot = _vmerge(
                    [jnp.minimum(bot[j], s_desc[j]) for j in range(MV)], False)
                return tuple(top) + tuple(bot)

            fin = lax.fori_loop(1, N // BATCH, body, tuple(top0) + tuple(bot0))
            for j in range(MV):
                out_v[pl.ds(LANES * j, LANES)] = fin[j]
                out_v[pl.ds(MV * LANES + LANES * j, LANES)] = fin[MV + j]
            pltpu.sync_copy(out_v, out_hbm.at[wid])

    return topk_kernel(emb)


def _sig(x):
    return 1.0 / (1.0 + jnp.exp(-x))


def _mlp_body(t_ref, cb_ref, w1_ref, b1_ref, w2_ref, b2_ref, w3_ref, b3_ref,
              o_ref):
    t = t_ref[...]
    pooled = jnp.concatenate(
        [t[:, :R], t[:, MV * LANES:MV * LANES + R]], axis=1) + cb_ref[0, 0]
    h = _sig(jnp.dot(pooled, w1_ref[...],
                     preferred_element_type=jnp.float32) + b1_ref[...])
    h = _sig(jnp.dot(h, w2_ref[...],
                     preferred_element_type=jnp.float32) + b2_ref[...])
    y = jnp.dot(h, w3_ref[...], preferred_element_type=jnp.float32) + b3_ref[...]
    o_ref[...] = _sig(y)


def _mlp(t, conv_b, W1, b1, W2, b2, W3, b3):
    B = t.shape[0]
    return pl.pallas_call(
        _mlp_body,
        in_specs=[
            pl.BlockSpec(t.shape, lambda: (0, 0)),
            pl.BlockSpec(memory_space=pltpu.SMEM),
            pl.BlockSpec(W1.shape, lambda: (0, 0)),
            pl.BlockSpec((1, b1.shape[1]), lambda: (0, 0)),
            pl.BlockSpec(W2.shape, lambda: (0, 0)),
            pl.BlockSpec((1, b2.shape[1]), lambda: (0, 0)),
            pl.BlockSpec(W3.shape, lambda: (0, 0)),
            pl.BlockSpec((1, b3.shape[1]), lambda: (0, 0)),
        ],
        out_specs=pl.BlockSpec((B, 1), lambda: (0, 0)),
        out_shape=jax.ShapeDtypeStruct((B, 1), jnp.float32),
    )(t, conv_b, W1, b1, W2, b2, W3, b3)


def kernel(x, conv_w, conv_b, W1, b1, W2, b2, W3, b3):
    B = x.shape[0]
    L = x.shape[-1]
    n_tiles = L // P
    x2 = x.reshape(B * n_tiles, P)
    w2 = conv_w.reshape(1, P)
    emb = _conv_embed(x2, w2, block_rows=1024).reshape(B, n_tiles)
    pooled = _topk_pool(emb)
    cb = conv_b.reshape(1, 1)
    return _mlp(pooled, cb, W1, b1.reshape(1, -1), W2, b2.reshape(1, -1),
                W3, b3.reshape(1, -1))


# speedup vs baseline: 1.0001x; 1.0001x over previous
import functools

import jax
import jax.numpy as jnp
from jax import lax
from jax.experimental import pallas as pl
from jax.experimental.pallas import tpu as pltpu
from jax.experimental.pallas import tpu_sc as plsc

R = 100
P = 2048
LANES = 16
MV = 8
BATCH = MV * LANES


def _conv_body(x_ref, w_ref, o_ref):
    x = x_ref[...]
    w = w_ref[...]
    acc = x[:, 0:128] * w[:, 0:128]
    for s in range(1, P // 128):
        acc = acc + x[:, s * 128:(s + 1) * 128] * w[:, s * 128:(s + 1) * 128]
    o_ref[...] = jnp.sum(acc, axis=1, keepdims=True)


def _conv_embed(x2, w2, block_rows):
    rows = x2.shape[0]
    grid = rows // block_rows
    return pl.pallas_call(
        _conv_body,
        grid=(grid,),
        in_specs=[
            pl.BlockSpec((block_rows, P), lambda i: (i, 0)),
            pl.BlockSpec((1, P), lambda i: (0, 0)),
        ],
        out_specs=pl.BlockSpec((block_rows, 1), lambda i: (i, 0)),
        out_shape=jax.ShapeDtypeStruct((rows, 1), jnp.float32),
    )(x2, w2)


def _s16(v, desc):
    k, _ = plsc.sort_key_val(v, v, descending=desc)
    return k


def _vmerge(vs, desc):
    if len(vs) == 1:
        return [_s16(vs[0], desc)]
    h = len(vs) // 2
    hi = [jnp.maximum(vs[i], vs[i + h]) for i in range(h)]
    lo = [jnp.minimum(vs[i], vs[i + h]) for i in range(h)]
    new = (hi + lo) if desc else (lo + hi)
    return _vmerge(new[:h], desc) + _vmerge(new[h:], desc)


def _bsort(vs, desc):
    if len(vs) == 1:
        return [_s16(vs[0], desc)]
    h = len(vs) // 2
    return _vmerge(_bsort(vs[:h], True) + _bsort(vs[h:], False), desc)


def _rev(vs):
    return [lax.rev(v, (0,)) for v in reversed(vs)]


def _topk_pool(emb):
    B, N = emb.shape
    out_w = 2 * MV * LANES
    mesh = plsc.VectorSubcoreMesh(core_axis_name="c", subcore_axis_name="s",
                                  num_cores=2, num_subcores=16)

    @functools.partial(
        pl.kernel,
        out_type=jax.ShapeDtypeStruct((B, out_w), jnp.float32),
        mesh=mesh,
        scratch_types=[
            pltpu.VMEM((N,), jnp.float32),
            pltpu.VMEM((out_w,), jnp.float32),
        ],
        compiler_params=pltpu.CompilerParams(needs_layout_passes=False),
    )
    def topk_kernel(emb_hbm, out_hbm, row_v, out_v):
        wid = lax.axis_index("s") * 2 + lax.axis_index("c")

        @pl.when(wid < B)
        def _():
            pltpu.sync_copy(emb_hbm.at[wid], row_v)

            def load(base):
                return [row_v[pl.ds(base + LANES * j, LANES)] for j in range(MV)]

            top0 = _bsort(load(0), True)
            bot0 = _rev(top0)

            def body(i, carry):
                top = list(carry[:MV])
                bot = list(carry[MV:])
                s_asc = _bsort(load(i * BATCH), False)
                s_desc = _rev(s_asc)
                top = _vmerge(
                    [jnp.maximum(top[j], s_asc[j]) for j in range(MV)], True)
                bot = _vmerge(
                    [jnp.minimum(bot[j], s_desc[j]) for j in range(MV)], False)
                return tuple(top) + tuple(bot)

            fin = lax.fori_loop(1, N // BATCH, body, tuple(top0) + tuple(bot0))
            for j in range(MV):
                out_v[pl.ds(LANES * j, LANES)] = fin[j]
                out_v[pl.ds(MV * LANES + LANES * j, LANES)] = fin[MV + j]
            pltpu.sync_copy(out_v, out_hbm.at[wid])

    return topk_kernel(emb)


def _sig(x):
    return 1.0 / (1.0 + jnp.exp(-x))


def _mlp_body(t_ref, cb_ref, w1_ref, b1_ref, w2_ref, b2_ref, w3_ref, b3_ref,
              o_ref):
    t = t_ref[...]
    pooled = jnp.concatenate(
        [t[:, :R], t[:, MV * LANES:MV * LANES + R]], axis=1) + cb_ref[0, 0]
    h = _sig(jnp.dot(pooled, w1_ref[...],
                     preferred_element_type=jnp.float32) + b1_ref[...])
    h = _sig(jnp.dot(h, w2_ref[...],
                     preferred_element_type=jnp.float32) + b2_ref[...])
    y = jnp.dot(h, w3_ref[...], preferred_element_type=jnp.float32) + b3_ref[...]
    o_ref[...] = _sig(y)


def _mlp(t, conv_b, W1, b1, W2, b2, W3, b3):
    B = t.shape[0]
    return pl.pallas_call(
        _mlp_body,
        in_specs=[
            pl.BlockSpec(t.shape, lambda: (0, 0)),
            pl.BlockSpec(memory_space=pltpu.SMEM),
            pl.BlockSpec(W1.shape, lambda: (0, 0)),
            pl.BlockSpec((1, b1.shape[1]), lambda: (0, 0)),
            pl.BlockSpec(W2.shape, lambda: (0, 0)),
            pl.BlockSpec((1, b2.shape[1]), lambda: (0, 0)),
            pl.BlockSpec(W3.shape, lambda: (0, 0)),
            pl.BlockSpec((1, b3.shape[1]), lambda: (0, 0)),
        ],
        out_specs=pl.BlockSpec((B, 1), lambda: (0, 0)),
        out_shape=jax.ShapeDtypeStruct((B, 1), jnp.float32),
    )(t, conv_b, W1, b1, W2, b2, W3, b3)


def kernel(x, conv_w, conv_b, W1, b1, W2, b2, W3, b3):
    B = x.shape[0]
    L = x.shape[-1]
    n_tiles = L // P
    x2 = x.reshape(B * n_tiles, P)
    w2 = conv_w.reshape(1, P)
    emb = _conv_embed(x2, w2, block_rows=1024).reshape(B, n_tiles)
    pooled = _topk_pool(emb)
    cb = conv_b.reshape(1, 1)
    return _mlp(pooled, cb, W1, b1.reshape(1, -1), W2, b2.reshape(1, -1),
                W3, b3.reshape(1, -1))
